# baseline (speedup 1.0000x reference)
import jax
import jax.numpy as jnp
from jax.experimental import pallas as pl
from jax.experimental.pallas import tpu as pltpu

N = 32768
NC = 2048
CD = 128
K = 8
BN = 512
EMB = 63
DIRCH = 27


def _body(qp_ref, emb_ref, dir_ref, cpt_ref, codes_ref,
          w0c_ref, w0e_ref, b0_ref, w1_ref, b1_ref,
          w2c_ref, w2e_ref, w2h_ref, b2_ref, w3_ref, b3_ref,
          wf_ref, bf_ref, wdf_ref, wdd_ref, bd_ref,
          ws_ref, bs_ref, wr_ref, br_ref, out_ref):
    f32 = jnp.float32
    qx = qp_ref[:, 0:1]
    qy = qp_ref[:, 1:2]
    qz = qp_ref[:, 2:3]
    cx = cpt_ref[0:1, :]
    cy = cpt_ref[1:2, :]
    cz = cpt_ref[2:3, :]
    dx = qx - cx
    dy = qy - cy
    dz = qz - cz
    d2 = dx * dx + dy * dy + dz * dz + 1e-16

    lane = jax.lax.broadcasted_iota(jnp.int32, (1, NC), 1)
    key = (jax.lax.bitcast_convert_type(d2, jnp.int32) & jnp.int32(-2048)) | lane

    wacc = jnp.zeros((BN, NC), f32)
    wsum = jnp.zeros((BN, 1), f32)
    imax = jnp.int32(2147483647)
    for _ in range(K):
        mk = jnp.min(key, axis=1, keepdims=True)
        hit = key == mk
        key = jnp.where(hit, imax, key)
        md = jax.lax.bitcast_convert_type(mk & jnp.int32(-2048), f32)
        r = jax.lax.rsqrt(md)
        wv = r * r * r
        wacc = jnp.where(hit, wv, wacc)
        wsum = wsum + wv

    qc = jnp.dot(wacc, codes_ref[...], preferred_element_type=f32)
    qc = qc * (1.0 / wsum)

    e = emb_ref[...]
    h = jnp.maximum(
        jnp.dot(qc, w0c_ref[...], preferred_element_type=f32)
        + jnp.dot(e, w0e_ref[...], preferred_element_type=f32)
        + b0_ref[...], 0.0)
    h = jnp.maximum(jnp.dot(h, w1_ref[...], preferred_element_type=f32)
                    + b1_ref[...], 0.0)
    h = jnp.maximum(
        jnp.dot(qc, w2c_ref[...], preferred_element_type=f32)
        + jnp.dot(e, w2e_ref[...], preferred_element_type=f32)
        + jnp.dot(h, w2h_ref[...], preferred_element_type=f32)
        + b2_ref[...], 0.0)
    h = jnp.maximum(jnp.dot(h, w3_ref[...], preferred_element_type=f32)
                    + b3_ref[...], 0.0)
    sigma = jnp.dot(h, ws_ref[...], preferred_element_type=f32) + bs_ref[...]
    final = jnp.dot(h, wf_ref[...], preferred_element_type=f32) + bf_ref[...]
    d = jnp.maximum(
        jnp.dot(final, wdf_ref[...], preferred_element_type=f32)
        + jnp.dot(dir_ref[...], wdd_ref[...], preferred_element_type=f32)
        + bd_ref[...], 0.0)
    rgb = jnp.dot(d, wr_ref[...], preferred_element_type=f32) + br_ref[...]
    out_ref[:, 0:3] = rgb
    out_ref[:, 3:4] = sigma


def kernel(indices, query_points, xyzdir_embedded, codes_position, codes,
           W0, b0, W1, b1, W2, b2, W3, b3, Wf, bf, Wd, bd, Ws, bs, Wr, br):
    idx0 = indices[0]
    cpos = jnp.take(codes_position, idx0, axis=0)
    cds = jnp.take(codes, idx0, axis=0)
    cpt = cpos.T
    emb = xyzdir_embedded[:, :EMB]
    dire = xyzdir_embedded[:, EMB:]

    w0c, w0e = W0[:CD], W0[CD:]
    w2c, w2e, w2h = W2[:CD], W2[CD:CD + EMB], W2[CD + EMB:]
    wdf, wdd = Wd[:CD], Wd[CD:]
    b0r = b0.reshape(1, -1)
    b1r = b1.reshape(1, -1)
    b2r = b2.reshape(1, -1)
    b3r = b3.reshape(1, -1)
    bfr = bf.reshape(1, -1)
    bdr = bd.reshape(1, -1)
    bsr = bs.reshape(1, -1)
    brr = br.reshape(1, -1)

    def full(shape):
        nd = len(shape)
        return pl.BlockSpec(shape, lambda i, nd=nd: (0,) * nd)

    row = lambda w: pl.BlockSpec((BN, w), lambda i: (i, 0))

    grid = (N // BN,)
    out = pl.pallas_call(
        _body,
        grid=grid,
        in_specs=[
            row(3), row(EMB), row(DIRCH),
            full((3, NC)), full((NC, CD)),
            full((CD, 128)), full((EMB, 128)), full((1, 128)),
            full((128, 128)), full((1, 128)),
            full((CD, 128)), full((EMB, 128)), full((128, 128)), full((1, 128)),
            full((128, 128)), full((1, 128)),
            full((128, 128)), full((1, 128)),
            full((128, 64)), full((DIRCH, 64)), full((1, 64)),
            full((128, 1)), full((1, 1)),
            full((64, 3)), full((1, 3)),
        ],
        out_specs=pl.BlockSpec((BN, 4), lambda i: (i, 0)),
        out_shape=jax.ShapeDtypeStruct((N, 4), jnp.float32),
        compiler_params=pltpu.CompilerParams(
            dimension_semantics=("arbitrary",),
        ),
    )(query_points, emb, dire, cpt, cds,
      w0c, w0e, b0r, W1, b1r,
      w2c, w2e, w2h, b2r, W3, b3r,
      Wf, bfr, wdf, wdd, bdr,
      Ws, bsr, Wr, brr)
    return out


# speedup vs baseline: 1.1816x; 1.1816x over previous
import jax
import jax.numpy as jnp
from jax.experimental import pallas as pl
from jax.experimental.pallas import tpu as pltpu

N = 32768
NC = 2048
CD = 128
K = 8
BN = 512
EMB = 63
DIRCH = 27


def _body(qp_ref, emb_ref, dir_ref, cpt_ref, codes_ref,
          w0c_ref, w0e_ref, b0_ref, w1_ref, b1_ref,
          w2c_ref, w2e_ref, w2h_ref, b2_ref, w3_ref, b3_ref,
          wf_ref, bf_ref, wdf_ref, wdd_ref, bd_ref,
          ws_ref, bs_ref, wr_ref, br_ref, out_ref):
    f32 = jnp.float32
    qx = qp_ref[:, 0:1]
    qy = qp_ref[:, 1:2]
    qz = qp_ref[:, 2:3]
    cx = cpt_ref[0:1, :]
    cy = cpt_ref[1:2, :]
    cz = cpt_ref[2:3, :]
    dx = qx - cx
    dy = qy - cy
    dz = qz - cz
    d2 = dx * dx + dy * dy + dz * dz + 1e-16

    lane = jax.lax.broadcasted_iota(jnp.int32, (1, NC), 1)
    keyi = (jax.lax.bitcast_convert_type(d2, jnp.int32) & jnp.int32(-2048)) | lane
    key = jax.lax.bitcast_convert_type(keyi, f32)

    wacc = jnp.zeros((BN, NC), f32)
    wsum = jnp.zeros((BN, 1), f32)
    for _ in range(K):
        mk = jnp.min(key, axis=1, keepdims=True)
        hit = key == mk
        key = jnp.where(hit, jnp.inf, key)
        mi = jax.lax.bitcast_convert_type(mk, jnp.int32)
        md = jax.lax.bitcast_convert_type(mi & jnp.int32(-2048), f32)
        r = jax.lax.rsqrt(md)
        wv = r * r * r
        wacc = jnp.where(hit, wv, wacc)
        wsum = wsum + wv

    qc = jnp.dot(wacc, codes_ref[...], preferred_element_type=f32)
    qc = qc * (1.0 / wsum)

    e = emb_ref[...]
    h = jnp.maximum(
        jnp.dot(qc, w0c_ref[...], preferred_element_type=f32)
        + jnp.dot(e, w0e_ref[...], preferred_element_type=f32)
        + b0_ref[...], 0.0)
    h = jnp.maximum(jnp.dot(h, w1_ref[...], preferred_element_type=f32)
                    + b1_ref[...], 0.0)
    h = jnp.maximum(
        jnp.dot(qc, w2c_ref[...], preferred_element_type=f32)
        + jnp.dot(e, w2e_ref[...], preferred_element_type=f32)
        + jnp.dot(h, w2h_ref[...], preferred_element_type=f32)
        + b2_ref[...], 0.0)
    h = jnp.maximum(jnp.dot(h, w3_ref[...], preferred_element_type=f32)
                    + b3_ref[...], 0.0)
    sigma = jnp.dot(h, ws_ref[...], preferred_element_type=f32) + bs_ref[...]
    final = jnp.dot(h, wf_ref[...], preferred_element_type=f32) + bf_ref[...]
    d = jnp.maximum(
        jnp.dot(final, wdf_ref[...], preferred_element_type=f32)
        + jnp.dot(dir_ref[...], wdd_ref[...], preferred_element_type=f32)
        + bd_ref[...], 0.0)
    rgb = jnp.dot(d, wr_ref[...], preferred_element_type=f32) + br_ref[...]
    out_ref[:, 0:3] = rgb
    out_ref[:, 3:4] = sigma


def kernel(indices, query_points, xyzdir_embedded, codes_position, codes,
           W0, b0, W1, b1, W2, b2, W3, b3, Wf, bf, Wd, bd, Ws, bs, Wr, br):
    idx0 = indices[0]
    cpos = jnp.take(codes_position, idx0, axis=0)
    cds = jnp.take(codes, idx0, axis=0)
    cpt = cpos.T
    emb = xyzdir_embedded[:, :EMB]
    dire = xyzdir_embedded[:, EMB:]

    w0c, w0e = W0[:CD], W0[CD:]
    w2c, w2e, w2h = W2[:CD], W2[CD:CD + EMB], W2[CD + EMB:]
    wdf, wdd = Wd[:CD], Wd[CD:]
    b0r = b0.reshape(1, -1)
    b1r = b1.reshape(1, -1)
    b2r = b2.reshape(1, -1)
    b3r = b3.reshape(1, -1)
    bfr = bf.reshape(1, -1)
    bdr = bd.reshape(1, -1)
    bsr = bs.reshape(1, -1)
    brr = br.reshape(1, -1)

    def full(shape):
        nd = len(shape)
        return pl.BlockSpec(shape, lambda i, nd=nd: (0,) * nd)

    row = lambda w: pl.BlockSpec((BN, w), lambda i: (i, 0))

    grid = (N // BN,)
    out = pl.pallas_call(
        _body,
        grid=grid,
        in_specs=[
            row(3), row(EMB), row(DIRCH),
            full((3, NC)), full((NC, CD)),
            full((CD, 128)), full((EMB, 128)), full((1, 128)),
            full((128, 128)), full((1, 128)),
            full((CD, 128)), full((EMB, 128)), full((128, 128)), full((1, 128)),
            full((128, 128)), full((1, 128)),
            full((128, 128)), full((1, 128)),
            full((128, 64)), full((DIRCH, 64)), full((1, 64)),
            full((128, 1)), full((1, 1)),
            full((64, 3)), full((1, 3)),
        ],
        out_specs=pl.BlockSpec((BN, 4), lambda i: (i, 0)),
        out_shape=jax.ShapeDtypeStruct((N, 4), jnp.float32),
        compiler_params=pltpu.CompilerParams(
            dimension_semantics=("arbitrary",),
        ),
    )(query_points, emb, dire, cpt, cds,
      w0c, w0e, b0r, W1, b1r,
      w2c, w2e, w2h, b2r, W3, b3r,
      Wf, bfr, wdf, wdd, bdr,
      Ws, bsr, Wr, brr)
    return out
